# baseline (speedup 1.0000x reference)
import functools

import jax
import jax.numpy as jnp
from jax import lax
from jax.experimental import pallas as pl
from jax.experimental.pallas import tpu as pltpu
from jax.experimental.pallas import tpu_sc as plsc

DIM = 32
SCALE = float(DIM ** 0.5)

NUM_CORES = 2
NUM_SUBCORES = 16
NW = NUM_CORES * NUM_SUBCORES

S0 = 16384
S1 = 50
BPW = S0 // NW
TC_BLK = BPW // 128

mesh = plsc.VectorSubcoreMesh(core_axis_name="c", subcore_axis_name="s")


@functools.partial(
    pl.kernel,
    mesh=mesh,
    out_type=jax.ShapeDtypeStruct((S1, DIM // 8, S0 // 128, 8, 128),
                                  jnp.float32),
    scratch_types=[
        pltpu.VMEM((S1, TC_BLK, 128), jnp.int32),
        pltpu.VMEM((2, BPW, DIM), jnp.float32),
        pltpu.VMEM((2, DIM, BPW), jnp.float32),
        pltpu.SemaphoreType.DMA,
        pltpu.SemaphoreType.DMA,
    ],
    compiler_params=pltpu.CompilerParams(
        use_tc_tiling_on_sc=False, needs_layout_passes=False),
)
def _emb(xt_hbm, table_hbm, out_hbm, idx_v, rows_v, trans_v, gsem, wsem):
    wid = lax.axis_index("s") * NUM_CORES + lax.axis_index("c")
    lanes = lax.iota(jnp.int32, 16)

    pltpu.sync_copy(xt_hbm.at[:, pl.ds(wid * TC_BLK, TC_BLK)], idx_v)

    def fire_gathers(d1, b):
        for j in range(TC_BLK):
            pltpu.async_copy(
                table_hbm.at[idx_v.at[d1, j]],
                rows_v.at[b, pl.ds(j * 128, 128)],
                gsem,
            )

    def drain_gathers(b):
        for j in range(TC_BLK):
            pltpu.make_async_copy(
                table_hbm.at[pl.ds(0, 128)],
                rows_v.at[b, pl.ds(j * 128, 128)],
                gsem,
            ).wait()

    def fire_writes(d1, b):
        for tr in range(DIM // 8):
            for t in range(TC_BLK):
                pltpu.async_copy(
                    trans_v.at[b, pl.ds(tr * 8, 8), pl.ds(t * 128, 128)],
                    out_hbm.at[d1, tr, wid * TC_BLK + t],
                    wsem,
                )

    def drain_writes(d1, b):
        for tr in range(DIM // 8):
            for t in range(TC_BLK):
                pltpu.make_async_copy(
                    trans_v.at[b, pl.ds(tr * 8, 8), pl.ds(t * 128, 128)],
                    out_hbm.at[d1, tr, wid * TC_BLK + t],
                    wsem,
                ).wait()

    fire_gathers(0, 0)

    def col_body(d1, carry):
        b = lax.rem(d1, 2)

        drain_gathers(b)

        @pl.when(d1 + 1 < S1)
        def _():
            fire_gathers(d1 + 1, 1 - b)

        @pl.when(d1 >= 2)
        def _():
            drain_writes(d1 - 2, b)

        def k_outer(k0, c):
            k_vec = jnp.bitwise_and(lanes + k0, DIM - 1)

            def b_body(bi, b_vec):
                v = plsc.load_gather(rows_v.at[b], [b_vec, k_vec])
                plsc.store_scatter(trans_v.at[b], [k_vec, b_vec], v * SCALE)
                return b_vec + 16

            lax.fori_loop(0, BPW // 16, b_body, lanes, unroll=8)
            return c

        lax.fori_loop(0, DIM, k_outer, 0)

        fire_writes(d1, b)
        return carry

    lax.fori_loop(0, S1, col_body, 0)
    drain_writes(S1 - 2, lax.rem(S1 - 2, 2))
    drain_writes(S1 - 1, lax.rem(S1 - 1, 2))


VOC = 1000000
FULL_COLS = (VOC // 128) * 128
CCHUNK = 768
N_CHUNKS = FULL_COLS // CCHUNK
BASE_LOCAL = N_CHUNKS // NW
TAIL = VOC - FULL_COLS


@functools.partial(
    pl.kernel,
    mesh=mesh,
    out_type=jax.ShapeDtypeStruct((VOC * DIM,), jnp.float32),
    scratch_types=[
        pltpu.VMEM((DIM, CCHUNK), jnp.float32),
        pltpu.VMEM((DIM, CCHUNK), jnp.float32),
        pltpu.VMEM((CCHUNK * DIM,), jnp.float32),
        pltpu.VMEM((CCHUNK * DIM,), jnp.float32),
        pltpu.SemaphoreType.DMA,
        pltpu.SemaphoreType.DMA,
    ],
    compiler_params=pltpu.CompilerParams(
        use_tc_tiling_on_sc=True, needs_layout_passes=False),
)
def _fmt(tt_hbm, tail_hbm, out_hbm, blk0, blk1, tr0, tr1, isem, wsem):
    wid = lax.axis_index("s") * NUM_CORES + lax.axis_index("c")
    lanes = lax.iota(jnp.int32, 16)
    n_local = BASE_LOCAL + jnp.where(wid < N_CHUNKS - BASE_LOCAL * NW, 1, 0)
    bufs = ((blk0, tr0), (blk1, tr1))

    def col_of(li):
        return (li * NW + wid) * CCHUNK

    def fire_load(li, blk):
        pltpu.async_copy(tt_hbm.at[:, pl.ds(col_of(li), CCHUNK)],
                         blk, isem)

    def drain_load(blk):
        pltpu.make_async_copy(tt_hbm.at[:, pl.ds(0, CCHUNK)],
                              blk, isem).wait()

    def fire_write(li, trs):
        pltpu.async_copy(trs,
                         out_hbm.at[pl.ds(col_of(li) * DIM, CCHUNK * DIM)],
                         wsem)

    def drain_write(trs):
        pltpu.make_async_copy(trs,
                              out_hbm.at[pl.ds(0, CCHUNK * DIM)],
                              wsem).wait()

    def transpose(blk, trs):
        def k_outer(k0, c):
            k_vec = jnp.bitwise_and(lanes + k0, DIM - 1)

            def c_body(ci, carry):
                c_vec, c_shift = carry
                v = plsc.load_gather(blk, [k_vec, c_vec])
                plsc.store_scatter(trs, [c_shift + k_vec], v)
                return (c_vec + 16, c_shift + 512)

            lax.fori_loop(0, CCHUNK // 16, c_body,
                          (lanes, jnp.left_shift(lanes, 5)), unroll=8)
            return c

        lax.fori_loop(0, DIM, k_outer, 0)

    fire_load(0, blk0)

    def pair_body(li2, carry):
        for b in range(2):
            blk, trs = bufs[b]
            li = li2 * 2 + b

            @pl.when(li < n_local)
            def _():
                drain_load(blk)

                @pl.when(li + 1 < n_local)
                def _():
                    fire_load(li + 1, bufs[1 - b][0])

                @pl.when(li >= 2)
                def _():
                    drain_write(trs)

                transpose(blk, trs)
                fire_write(li, trs)
        return carry

    lax.fori_loop(0, (BASE_LOCAL + 2) // 2, pair_body, 0)

    drain_write(tr0)
    drain_write(tr1)

    @pl.when(wid == 0)
    def _():
        pltpu.sync_copy(tail_hbm, tr0.at[pl.ds(0, TAIL * DIM)])
        pltpu.sync_copy(tr0.at[pl.ds(0, TAIL * DIM)],
                        out_hbm.at[pl.ds(FULL_COLS * DIM, TAIL * DIM)])


def kernel(x, table):
    xt = x.T.astype(jnp.int32).reshape(S1, S0 // 128, 128)
    tail_flat = table[FULL_COLS:].reshape(TAIL * DIM)
    flat_table = _fmt(table.T, tail_flat)
    a = _emb(xt, flat_table.reshape(VOC, DIM))
    return a.transpose(2, 4, 0, 1, 3).reshape(S0, S1, DIM)


# speedup vs baseline: 1.0308x; 1.0308x over previous
import functools

import jax
import jax.numpy as jnp
from jax import lax
from jax.experimental import pallas as pl
from jax.experimental.pallas import tpu as pltpu
from jax.experimental.pallas import tpu_sc as plsc

DIM = 32
SCALE = float(DIM ** 0.5)

NUM_CORES = 2
NUM_SUBCORES = 16
NW = NUM_CORES * NUM_SUBCORES

S0 = 16384
S1 = 50
BPW = S0 // NW
TC_BLK = BPW // 128

mesh = plsc.VectorSubcoreMesh(core_axis_name="c", subcore_axis_name="s")


@functools.partial(
    pl.kernel,
    mesh=mesh,
    out_type=jax.ShapeDtypeStruct((S1, DIM // 8, S0 // 128, 8, 128),
                                  jnp.float32),
    scratch_types=[
        pltpu.VMEM((S1, TC_BLK, 128), jnp.int32),
        pltpu.VMEM((2, BPW, DIM), jnp.float32),
        pltpu.VMEM((2, DIM, BPW), jnp.float32),
        pltpu.SemaphoreType.DMA,
        pltpu.SemaphoreType.DMA,
    ],
    compiler_params=pltpu.CompilerParams(
        use_tc_tiling_on_sc=False, needs_layout_passes=False),
)
def _emb(xt_hbm, table_hbm, out_hbm, idx_v, rows_v, trans_v, gsem, wsem):
    wid = lax.axis_index("s") * NUM_CORES + lax.axis_index("c")
    lanes = lax.iota(jnp.int32, 16)

    pltpu.sync_copy(xt_hbm.at[:, pl.ds(wid * TC_BLK, TC_BLK)], idx_v)

    def fire_gathers(d1, b):
        for j in range(TC_BLK):
            pltpu.async_copy(
                table_hbm.at[idx_v.at[d1, j]],
                rows_v.at[b, pl.ds(j * 128, 128)],
                gsem,
            )

    def drain_gathers(b):
        for j in range(TC_BLK):
            pltpu.make_async_copy(
                table_hbm.at[pl.ds(0, 128)],
                rows_v.at[b, pl.ds(j * 128, 128)],
                gsem,
            ).wait()

    def fire_writes(d1, b):
        for tr in range(DIM // 8):
            for t in range(TC_BLK):
                pltpu.async_copy(
                    trans_v.at[b, pl.ds(tr * 8, 8), pl.ds(t * 128, 128)],
                    out_hbm.at[d1, tr, wid * TC_BLK + t],
                    wsem,
                )

    def drain_writes(d1, b):
        for tr in range(DIM // 8):
            for t in range(TC_BLK):
                pltpu.make_async_copy(
                    trans_v.at[b, pl.ds(tr * 8, 8), pl.ds(t * 128, 128)],
                    out_hbm.at[d1, tr, wid * TC_BLK + t],
                    wsem,
                ).wait()

    fire_gathers(0, 0)

    def col_body(d1, carry):
        b = lax.rem(d1, 2)

        drain_gathers(b)

        @pl.when(d1 + 1 < S1)
        def _():
            fire_gathers(d1 + 1, 1 - b)

        @pl.when(d1 >= 2)
        def _():
            drain_writes(d1 - 2, b)

        def k_outer(k0, c):
            k_vec = jnp.bitwise_and(lanes + k0, DIM - 1)

            def b_body(bi, b_vec):
                v = plsc.load_gather(rows_v.at[b], [b_vec, k_vec])
                plsc.store_scatter(trans_v.at[b], [k_vec, b_vec], v * SCALE)
                return b_vec + 16

            lax.fori_loop(0, BPW // 16, b_body, lanes, unroll=8)
            return c

        lax.fori_loop(0, DIM, k_outer, 0)

        fire_writes(d1, b)
        return carry

    lax.fori_loop(0, S1, col_body, 0)
    drain_writes(S1 - 2, lax.rem(S1 - 2, 2))
    drain_writes(S1 - 1, lax.rem(S1 - 1, 2))


VOC = 1000000
FULL_COLS = (VOC // 128) * 128
CCHUNK = 512
N_CHUNKS = FULL_COLS // CCHUNK
BASE_LOCAL = N_CHUNKS // NW
TAIL = VOC - FULL_COLS


@functools.partial(
    pl.kernel,
    mesh=mesh,
    out_type=jax.ShapeDtypeStruct((VOC * DIM,), jnp.float32),
    scratch_types=[
        pltpu.VMEM((DIM, CCHUNK), jnp.float32),
        pltpu.VMEM((DIM, CCHUNK), jnp.float32),
        pltpu.VMEM((CCHUNK * DIM,), jnp.float32),
        pltpu.VMEM((CCHUNK * DIM,), jnp.float32),
        pltpu.SemaphoreType.DMA,
        pltpu.SemaphoreType.DMA,
    ],
    compiler_params=pltpu.CompilerParams(
        use_tc_tiling_on_sc=True, needs_layout_passes=False),
)
def _fmt(tt_hbm, tail_hbm, out_hbm, blk0, blk1, tr0, tr1, isem, wsem):
    wid = lax.axis_index("s") * NUM_CORES + lax.axis_index("c")
    lanes = lax.iota(jnp.int32, 16)
    n_local = BASE_LOCAL + jnp.where(wid < N_CHUNKS - BASE_LOCAL * NW, 1, 0)
    bufs = ((blk0, tr0), (blk1, tr1))

    def col_of(li):
        return (li * NW + wid) * CCHUNK

    def fire_load(li, blk):
        pltpu.async_copy(tt_hbm.at[:, pl.ds(col_of(li), CCHUNK)],
                         blk, isem)

    def drain_load(blk):
        pltpu.make_async_copy(tt_hbm.at[:, pl.ds(0, CCHUNK)],
                              blk, isem).wait()

    def fire_write(li, trs):
        pltpu.async_copy(trs,
                         out_hbm.at[pl.ds(col_of(li) * DIM, CCHUNK * DIM)],
                         wsem)

    def drain_write(trs):
        pltpu.make_async_copy(trs,
                              out_hbm.at[pl.ds(0, CCHUNK * DIM)],
                              wsem).wait()

    def transpose(blk, trs):
        def k_outer(k0, c):
            k_vec = jnp.bitwise_and(lanes + k0, DIM - 1)

            def c_body(ci, carry):
                c_vec, c_shift = carry
                v = plsc.load_gather(blk, [k_vec, c_vec])
                plsc.store_scatter(trs, [c_shift + k_vec], v)
                return (c_vec + 16, c_shift + 512)

            lax.fori_loop(0, CCHUNK // 16, c_body,
                          (lanes, jnp.left_shift(lanes, 5)), unroll=8)
            return c

        lax.fori_loop(0, DIM, k_outer, 0)

    fire_load(0, blk0)

    def pair_body(li2, carry):
        for b in range(2):
            blk, trs = bufs[b]
            li = li2 * 2 + b

            @pl.when(li < n_local)
            def _():
                drain_load(blk)

                @pl.when(li + 1 < n_local)
                def _():
                    fire_load(li + 1, bufs[1 - b][0])

                @pl.when(li >= 2)
                def _():
                    drain_write(trs)

                transpose(blk, trs)
                fire_write(li, trs)
        return carry

    lax.fori_loop(0, (BASE_LOCAL + 2) // 2, pair_body, 0)

    drain_write(tr0)
    drain_write(tr1)

    @pl.when(wid == 0)
    def _():
        pltpu.sync_copy(tail_hbm, tr0.at[pl.ds(0, TAIL * DIM)])
        pltpu.sync_copy(tr0.at[pl.ds(0, TAIL * DIM)],
                        out_hbm.at[pl.ds(FULL_COLS * DIM, TAIL * DIM)])


def kernel(x, table):
    xt = x.T.astype(jnp.int32).reshape(S1, S0 // 128, 128)
    tail_flat = table[FULL_COLS:].reshape(TAIL * DIM)
    flat_table = _fmt(table.T, tail_flat)
    a = _emb(xt, flat_table.reshape(VOC, DIM))
    return a.transpose(2, 4, 0, 1, 3).reshape(S0, S1, DIM)
